# baseline (speedup 1.0000x reference)
import functools
import math

import jax
import jax.numpy as jnp
from jax import lax
from jax.experimental import pallas as pl
from jax.experimental.pallas import tpu as pltpu
from jax.experimental.pallas import tpu_sc as plsc

_EPS = 1e-12
_RB = 768
_CB = 4096
_LANES = 128

_NC = 2
_NS = 16
_NW = _NC * _NS


def _argmin_body(tm2_ref, c_ref, tsq_ref, csq_ref, val_ref, idx_ref,
                 s0_ref, s1_ref, rv_ref, ri_ref, *, nblk):
    jc = pl.program_id(1)

    def matmul(sw_ref):
        sw_ref[...] = lax.dot_general(
            tm2_ref[...], c_ref[...], (((1,), (1,)), ((), ())),
            preferred_element_type=jnp.float32,
            precision=lax.Precision.DEFAULT)

    def argmin(sr_ref, g):
        tsq = tsq_ref[...]
        nch = _CB // _LANES
        lv = li = None
        for ch in range(nch):
            s = sr_ref[:, ch * _LANES:(ch + 1) * _LANES]
            csq = csq_ref[:, pl.ds(g * _CB + ch * _LANES, _LANES)]
            d = (tsq + csq) + s
            j = lax.broadcasted_iota(jnp.int32, d.shape, 1) + (
                g * _CB + ch * _LANES)
            if lv is None:
                lv, li = d, j
            else:
                upd = d < lv
                lv = jnp.where(upd, d, lv)
                li = jnp.where(upd, j, li)
        upd = lv < rv_ref[...]
        rv_ref[...] = jnp.where(upd, lv, rv_ref[...])
        ri_ref[...] = jnp.where(upd, li, ri_ref[...])

    @pl.when(jc == 0)
    def _prologue():
        rv_ref[...] = jnp.full((_RB, _LANES), jnp.inf, jnp.float32)
        ri_ref[...] = jnp.zeros((_RB, _LANES), jnp.int32)
        matmul(s0_ref)

    @pl.when(jc % 2 == 1)
    def _odd():
        matmul(s1_ref)
        argmin(s0_ref, jc - 1)

    @pl.when(jnp.logical_and(jc % 2 == 0, jc > 0))
    def _even():
        @pl.when(jc < nblk)
        def _():
            matmul(s0_ref)
        argmin(s1_ref, jc - 1)

    @pl.when(jc == nblk)
    def _finalize():
        rv = rv_ref[...]
        ri = ri_ref[...]
        m = jnp.min(rv, axis=1, keepdims=True)
        cand = jnp.where(rv == m, ri, jnp.int32(2 ** 30))
        val_ref[...] = m
        idx_ref[...] = jnp.min(cand, axis=1, keepdims=True)


def _argmin_call(t, code, tsq, csq):
    rows, cdim = t.shape
    book = code.shape[0]
    nblk = book // _CB
    grid = (rows // _RB, nblk + 1)
    _, idx = pl.pallas_call(
        functools.partial(_argmin_body, nblk=nblk),
        grid=grid,
        in_specs=[
            pl.BlockSpec((_RB, cdim), lambda i, j: (i, 0)),
            pl.BlockSpec((_CB, cdim),
                         lambda i, j, n=nblk: (jnp.minimum(j, n - 1), 0)),
            pl.BlockSpec((_RB, 1), lambda i, j: (i, 0)),
            pl.BlockSpec((1, book), lambda i, j: (0, 0)),
        ],
        out_specs=[
            pl.BlockSpec((_RB, 1), lambda i, j: (i, 0)),
            pl.BlockSpec((_RB, 1), lambda i, j: (i, 0)),
        ],
        out_shape=[
            jax.ShapeDtypeStruct((rows, 1), jnp.float32),
            jax.ShapeDtypeStruct((rows, 1), jnp.int32),
        ],
        scratch_shapes=[
            pltpu.VMEM((_RB, _CB), jnp.float32),
            pltpu.VMEM((_RB, _CB), jnp.float32),
            pltpu.VMEM((_RB, _LANES), jnp.float32),
            pltpu.VMEM((_RB, _LANES), jnp.int32),
        ],
    )(t, code, tsq, csq)
    return idx.reshape(rows)


def _l2n(x):
    n = jnp.sqrt(jnp.sum(x * x, axis=-1, keepdims=True))
    return x / jnp.maximum(n, _EPS)


@functools.lru_cache(maxsize=None)
def _make_gather(nrows_tbl, width, rows, out_width):
    bpw = rows // _NW
    mesh = plsc.VectorSubcoreMesh(core_axis_name="c", subcore_axis_name="s")

    @functools.partial(
        pl.kernel,
        mesh=mesh,
        out_type=jax.ShapeDtypeStruct((rows, out_width), jnp.float32),
        scratch_types=[
            pltpu.VMEM((bpw,), jnp.int32),
            pltpu.VMEM((bpw, width), jnp.float32),
            pltpu.SemaphoreType.DMA,
        ],
    )
    def gather_kernel(table_hbm, idx_hbm, out_hbm, idx_v, rows_v, sem):
        wid = lax.axis_index("s") * _NC + lax.axis_index("c")
        base = wid * bpw
        pltpu.sync_copy(idx_hbm.at[pl.ds(base, bpw)], idx_v)
        pltpu.async_copy(table_hbm.at[idx_v], rows_v, sem).wait()
        if out_width == width:
            pltpu.sync_copy(rows_v, out_hbm.at[pl.ds(base, bpw)])
        else:
            pltpu.sync_copy(rows_v.at[:, 0:out_width],
                            out_hbm.at[pl.ds(base, bpw)])

    return gather_kernel


def _fuse_body(cls_ref, new_ref, k_ref, q_ref, v_ref, w_ref, b_ref, o_ref):
    C = cls_ref[...]
    N = new_ref[...]
    dim = C.shape[1]
    dn = (((1,), (1,)), ((), ()))

    def dot(a, b):
        return lax.dot_general(a, b, dn,
                               preferred_element_type=jnp.float32,
                               precision=lax.Precision.DEFAULT)

    Kc, Kn = dot(C, k_ref[...]), dot(N, k_ref[...])
    Qc, Qn = dot(C, q_ref[...]), dot(N, q_ref[...])
    Vc, Vn = dot(C, v_ref[...]), dot(N, v_ref[...])
    Kcs = Kc.astype(jnp.bfloat16).astype(jnp.float32)
    Kns = Kn.astype(jnp.bfloat16).astype(jnp.float32)
    Qcs = Qc.astype(jnp.bfloat16).astype(jnp.float32)
    Qns = Qn.astype(jnp.bfloat16).astype(jnp.float32)
    sq2 = jnp.float32(math.sqrt(2.0))
    s00 = jnp.sum(Kcs * Qcs, axis=1, keepdims=True) / sq2
    s01 = jnp.sum(Kcs * Qns, axis=1, keepdims=True) / sq2
    s10 = jnp.sum(Kns * Qcs, axis=1, keepdims=True) / sq2
    s11 = jnp.sum(Kns * Qns, axis=1, keepdims=True) / sq2
    m0 = jnp.maximum(s00, s10)
    e00 = jnp.exp(s00 - m0)
    e10 = jnp.exp(s10 - m0)
    d0 = e00 + e10
    m1 = jnp.maximum(s01, s11)
    e01 = jnp.exp(s01 - m1)
    e11 = jnp.exp(s11 - m1)
    d1 = e01 + e11
    f0 = (e00 / d0) * Vc + (e10 / d0) * Vn
    f1 = (e01 / d1) * Vc + (e11 / d1) * Vn
    w1 = w_ref[0:dim, :]
    w2 = w_ref[dim:2 * dim, :]
    dn2 = (((1,), (0,)), ((), ()))
    out = (lax.dot_general(f0, w1, dn2, preferred_element_type=jnp.float32,
                           precision=lax.Precision.DEFAULT)
           + lax.dot_general(f1, w2, dn2, preferred_element_type=jnp.float32,
                             precision=lax.Precision.DEFAULT)
           + b_ref[...])
    o_ref[:, 0, :] = out * (1.0 / (1.0 + jnp.exp(-1.702 * out)))


def _fuse_call(cls, new, K, Q, V, W_fuse, b2):
    bs, dim = cls.shape
    return pl.pallas_call(
        _fuse_body,
        out_shape=jax.ShapeDtypeStruct((bs, 1, dim), jnp.float32),
    )(cls, new, K, Q, V, W_fuse, b2)


def kernel(tokens, code_book, K, Q, V, W_fuse, b_fuse):
    bs, _, dim = tokens.shape
    book, cdim = code_book.shape
    cls = tokens[:, 0, :]
    t2 = cls.reshape(-1, cdim)
    t = _l2n(t2)
    code = _l2n(code_book)
    tsq = jnp.sum(t ** 2, axis=1, keepdims=True)
    csq = jnp.sum(code ** 2, axis=1)[None, :]
    idx = _argmin_call(-2.0 * t, code, tsq, csq)
    table = jnp.concatenate([code_book, code_book], axis=1)
    pair = _make_gather(book, 2 * cdim, t2.shape[0], 2 * cdim)(table, idx)
    new = pair[:, :cdim].reshape(bs, dim)
    return _fuse_call(cls, new, K, Q, V, W_fuse, b_fuse.reshape(1, dim))


# speedup vs baseline: 1.1157x; 1.1157x over previous
import functools
import math

import jax
import jax.numpy as jnp
from jax import lax
from jax.experimental import pallas as pl
from jax.experimental.pallas import tpu as pltpu
from jax.experimental.pallas import tpu_sc as plsc

_EPS = 1e-12
_RB = 768
_CB = 2048
_LANES = 128

_NC = 2
_NS = 16
_NW = _NC * _NS


def _argmin_body(tm2_ref, c_ref, tsq_ref, csq_ref, val_ref, idx_ref,
                 s0_ref, s1_ref, rv_ref, ri_ref, *, nblk):
    jc = pl.program_id(1)

    def matmul(sw_ref):
        sw_ref[...] = lax.dot_general(
            tm2_ref[...], c_ref[...], (((1,), (1,)), ((), ())),
            preferred_element_type=jnp.float32,
            precision=lax.Precision.DEFAULT)

    def argmin(sr_ref, g):
        tsq = tsq_ref[...]
        nch = _CB // _LANES
        lv = li = None
        for ch in range(nch):
            s = sr_ref[:, ch * _LANES:(ch + 1) * _LANES]
            csq = csq_ref[:, pl.ds(g * _CB + ch * _LANES, _LANES)]
            d = (tsq + csq) + s
            j = lax.broadcasted_iota(jnp.int32, d.shape, 1) + (
                g * _CB + ch * _LANES)
            if lv is None:
                lv, li = d, j
            else:
                upd = d < lv
                lv = jnp.where(upd, d, lv)
                li = jnp.where(upd, j, li)
        upd = lv < rv_ref[...]
        rv_ref[...] = jnp.where(upd, lv, rv_ref[...])
        ri_ref[...] = jnp.where(upd, li, ri_ref[...])

    @pl.when(jc == 0)
    def _prologue():
        rv_ref[...] = jnp.full((_RB, _LANES), jnp.inf, jnp.float32)
        ri_ref[...] = jnp.zeros((_RB, _LANES), jnp.int32)
        matmul(s0_ref)

    @pl.when(jc % 2 == 1)
    def _odd():
        matmul(s1_ref)
        argmin(s0_ref, jc - 1)

    @pl.when(jnp.logical_and(jc % 2 == 0, jc > 0))
    def _even():
        @pl.when(jc < nblk)
        def _():
            matmul(s0_ref)
        argmin(s1_ref, jc - 1)

    @pl.when(jc == nblk)
    def _finalize():
        rv = rv_ref[...]
        ri = ri_ref[...]
        m = jnp.min(rv, axis=1, keepdims=True)
        cand = jnp.where(rv == m, ri, jnp.int32(2 ** 30))
        val_ref[...] = m
        idx_ref[...] = jnp.min(cand, axis=1, keepdims=True)


def _argmin_call(t, code, tsq, csq):
    rows, cdim = t.shape
    book = code.shape[0]
    nblk = book // _CB
    grid = (rows // _RB, nblk + 1)
    _, idx = pl.pallas_call(
        functools.partial(_argmin_body, nblk=nblk),
        grid=grid,
        in_specs=[
            pl.BlockSpec((_RB, cdim), lambda i, j: (i, 0)),
            pl.BlockSpec((_CB, cdim),
                         lambda i, j, n=nblk: (jnp.minimum(j, n - 1), 0)),
            pl.BlockSpec((_RB, 1), lambda i, j: (i, 0)),
            pl.BlockSpec((1, book), lambda i, j: (0, 0)),
        ],
        out_specs=[
            pl.BlockSpec((_RB, 1), lambda i, j: (i, 0)),
            pl.BlockSpec((_RB, 1), lambda i, j: (i, 0)),
        ],
        out_shape=[
            jax.ShapeDtypeStruct((rows, 1), jnp.float32),
            jax.ShapeDtypeStruct((rows, 1), jnp.int32),
        ],
        scratch_shapes=[
            pltpu.VMEM((_RB, _CB), jnp.float32),
            pltpu.VMEM((_RB, _CB), jnp.float32),
            pltpu.VMEM((_RB, _LANES), jnp.float32),
            pltpu.VMEM((_RB, _LANES), jnp.int32),
        ],
    )(t, code, tsq, csq)
    return idx.reshape(rows)


def _l2n(x):
    n = jnp.sqrt(jnp.sum(x * x, axis=-1, keepdims=True))
    return x / jnp.maximum(n, _EPS)


@functools.lru_cache(maxsize=None)
def _make_gather(nrows_tbl, width, rows, out_width):
    bpw = rows // _NW
    mesh = plsc.VectorSubcoreMesh(core_axis_name="c", subcore_axis_name="s")

    @functools.partial(
        pl.kernel,
        mesh=mesh,
        out_type=jax.ShapeDtypeStruct((rows, out_width), jnp.float32),
        scratch_types=[
            pltpu.VMEM((bpw,), jnp.int32),
            pltpu.VMEM((bpw, width), jnp.float32),
            pltpu.SemaphoreType.DMA,
        ],
    )
    def gather_kernel(table_hbm, idx_hbm, out_hbm, idx_v, rows_v, sem):
        wid = lax.axis_index("s") * _NC + lax.axis_index("c")
        base = wid * bpw
        pltpu.sync_copy(idx_hbm.at[pl.ds(base, bpw)], idx_v)
        pltpu.async_copy(table_hbm.at[idx_v], rows_v, sem).wait()
        if out_width == width:
            pltpu.sync_copy(rows_v, out_hbm.at[pl.ds(base, bpw)])
        else:
            pltpu.sync_copy(rows_v.at[:, 0:out_width],
                            out_hbm.at[pl.ds(base, bpw)])

    return gather_kernel


def _fuse_body(cls_ref, new_ref, k_ref, q_ref, v_ref, w_ref, b_ref, o_ref):
    C = cls_ref[...]
    N = new_ref[...]
    dim = C.shape[1]
    dn = (((1,), (1,)), ((), ()))

    def dot(a, b):
        return lax.dot_general(a, b, dn,
                               preferred_element_type=jnp.float32,
                               precision=lax.Precision.DEFAULT)

    Kc, Kn = dot(C, k_ref[...]), dot(N, k_ref[...])
    Qc, Qn = dot(C, q_ref[...]), dot(N, q_ref[...])
    Vc, Vn = dot(C, v_ref[...]), dot(N, v_ref[...])
    Kcs = Kc.astype(jnp.bfloat16).astype(jnp.float32)
    Kns = Kn.astype(jnp.bfloat16).astype(jnp.float32)
    Qcs = Qc.astype(jnp.bfloat16).astype(jnp.float32)
    Qns = Qn.astype(jnp.bfloat16).astype(jnp.float32)
    sq2 = jnp.float32(math.sqrt(2.0))
    s00 = jnp.sum(Kcs * Qcs, axis=1, keepdims=True) / sq2
    s01 = jnp.sum(Kcs * Qns, axis=1, keepdims=True) / sq2
    s10 = jnp.sum(Kns * Qcs, axis=1, keepdims=True) / sq2
    s11 = jnp.sum(Kns * Qns, axis=1, keepdims=True) / sq2
    m0 = jnp.maximum(s00, s10)
    e00 = jnp.exp(s00 - m0)
    e10 = jnp.exp(s10 - m0)
    d0 = e00 + e10
    m1 = jnp.maximum(s01, s11)
    e01 = jnp.exp(s01 - m1)
    e11 = jnp.exp(s11 - m1)
    d1 = e01 + e11
    f0 = (e00 / d0) * Vc + (e10 / d0) * Vn
    f1 = (e01 / d1) * Vc + (e11 / d1) * Vn
    w1 = w_ref[0:dim, :]
    w2 = w_ref[dim:2 * dim, :]
    dn2 = (((1,), (0,)), ((), ()))
    out = (lax.dot_general(f0, w1, dn2, preferred_element_type=jnp.float32,
                           precision=lax.Precision.DEFAULT)
           + lax.dot_general(f1, w2, dn2, preferred_element_type=jnp.float32,
                             precision=lax.Precision.DEFAULT)
           + b_ref[...])
    o_ref[:, 0, :] = out * (1.0 / (1.0 + jnp.exp(-1.702 * out)))


def _fuse_call(cls, new, K, Q, V, W_fuse, b2):
    bs, dim = cls.shape
    return pl.pallas_call(
        _fuse_body,
        out_shape=jax.ShapeDtypeStruct((bs, 1, dim), jnp.float32),
    )(cls, new, K, Q, V, W_fuse, b2)


def kernel(tokens, code_book, K, Q, V, W_fuse, b_fuse):
    bs, _, dim = tokens.shape
    book, cdim = code_book.shape
    cls = tokens[:, 0, :]
    t2 = cls.reshape(-1, cdim)
    t = _l2n(t2)
    code = _l2n(code_book)
    tsq = jnp.sum(t ** 2, axis=1, keepdims=True)
    csq = jnp.sum(code ** 2, axis=1)[None, :]
    idx = _argmin_call(-2.0 * t, code, tsq, csq)
    table = jnp.concatenate([code_book, code_book], axis=1)
    pair = _make_gather(book, 2 * cdim, t2.shape[0], 2 * cdim)(table, idx)
    new = pair[:, :cdim].reshape(bs, dim)
    return _fuse_call(cls, new, K, Q, V, W_fuse, b_fuse.reshape(1, dim))


# speedup vs baseline: 1.2296x; 1.1021x over previous
import functools
import math

import jax
import jax.numpy as jnp
from jax import lax
from jax.experimental import pallas as pl
from jax.experimental.pallas import tpu as pltpu
from jax.experimental.pallas import tpu_sc as plsc

_EPS = 1e-12
_RB = 768
_CB = 2048
_LANES = 128

_NC = 2
_NS = 16
_NW = _NC * _NS


def _argmin_body(tm2_ref, c_ref, tsq_ref, csq_ref, val_ref, idx_ref,
                 rv_ref, ri_ref, *, nblk):
    jc = pl.program_id(1)

    def chunked(g):
        tsq = tsq_ref[...]
        nch = _CB // _LANES
        lv = li = None
        for ch in range(nch):
            s = lax.dot_general(
                tm2_ref[...], c_ref[ch * _LANES:(ch + 1) * _LANES, :],
                (((1,), (1,)), ((), ())),
                preferred_element_type=jnp.float32,
                precision=lax.Precision.DEFAULT)
            csq = csq_ref[:, pl.ds(g * _CB + ch * _LANES, _LANES)]
            d = (tsq + csq) + s
            j = lax.broadcasted_iota(jnp.int32, d.shape, 1) + (
                g * _CB + ch * _LANES)
            if lv is None:
                lv, li = d, j
            else:
                upd = d < lv
                lv = jnp.where(upd, d, lv)
                li = jnp.where(upd, j, li)
        upd = lv < rv_ref[...]
        rv_ref[...] = jnp.where(upd, lv, rv_ref[...])
        ri_ref[...] = jnp.where(upd, li, ri_ref[...])

    @pl.when(jc == 0)
    def _prologue():
        rv_ref[...] = jnp.full((_RB, _LANES), jnp.inf, jnp.float32)
        ri_ref[...] = jnp.zeros((_RB, _LANES), jnp.int32)

    chunked(jc)

    @pl.when(jc == nblk - 1)
    def _finalize():
        rv = rv_ref[...]
        ri = ri_ref[...]
        m = jnp.min(rv, axis=1, keepdims=True)
        cand = jnp.where(rv == m, ri, jnp.int32(2 ** 30))
        val_ref[...] = m
        idx_ref[...] = jnp.min(cand, axis=1, keepdims=True)


def _argmin_call(t, code, tsq, csq):
    rows, cdim = t.shape
    book = code.shape[0]
    nblk = book // _CB
    grid = (rows // _RB, nblk)
    _, idx = pl.pallas_call(
        functools.partial(_argmin_body, nblk=nblk),
        grid=grid,
        in_specs=[
            pl.BlockSpec((_RB, cdim), lambda i, j: (i, 0)),
            pl.BlockSpec((_CB, cdim), lambda i, j: (j, 0)),
            pl.BlockSpec((_RB, 1), lambda i, j: (i, 0)),
            pl.BlockSpec((1, book), lambda i, j: (0, 0)),
        ],
        out_specs=[
            pl.BlockSpec((_RB, 1), lambda i, j: (i, 0)),
            pl.BlockSpec((_RB, 1), lambda i, j: (i, 0)),
        ],
        out_shape=[
            jax.ShapeDtypeStruct((rows, 1), jnp.float32),
            jax.ShapeDtypeStruct((rows, 1), jnp.int32),
        ],
        scratch_shapes=[
            pltpu.VMEM((_RB, _LANES), jnp.float32),
            pltpu.VMEM((_RB, _LANES), jnp.int32),
        ],
    )(t, code, tsq, csq)
    return idx.reshape(rows)


def _l2n(x):
    n = jnp.sqrt(jnp.sum(x * x, axis=-1, keepdims=True))
    return x / jnp.maximum(n, _EPS)


@functools.lru_cache(maxsize=None)
def _make_gather(nrows_tbl, width, rows, out_width):
    bpw = rows // _NW
    mesh = plsc.VectorSubcoreMesh(core_axis_name="c", subcore_axis_name="s")

    @functools.partial(
        pl.kernel,
        mesh=mesh,
        out_type=jax.ShapeDtypeStruct((rows, out_width), jnp.float32),
        scratch_types=[
            pltpu.VMEM((bpw,), jnp.int32),
            pltpu.VMEM((bpw, width), jnp.float32),
            pltpu.SemaphoreType.DMA,
        ],
    )
    def gather_kernel(table_hbm, idx_hbm, out_hbm, idx_v, rows_v, sem):
        wid = lax.axis_index("s") * _NC + lax.axis_index("c")
        base = wid * bpw
        pltpu.sync_copy(idx_hbm.at[pl.ds(base, bpw)], idx_v)
        pltpu.async_copy(table_hbm.at[idx_v], rows_v, sem).wait()
        if out_width == width:
            pltpu.sync_copy(rows_v, out_hbm.at[pl.ds(base, bpw)])
        else:
            pltpu.sync_copy(rows_v.at[:, 0:out_width],
                            out_hbm.at[pl.ds(base, bpw)])

    return gather_kernel


def _fuse_body(cls_ref, new_ref, k_ref, q_ref, v_ref, w_ref, b_ref, o_ref):
    C = cls_ref[...]
    N = new_ref[...]
    dim = C.shape[1]
    dn = (((1,), (1,)), ((), ()))

    def dot(a, b):
        return lax.dot_general(a, b, dn,
                               preferred_element_type=jnp.float32,
                               precision=lax.Precision.DEFAULT)

    Kc, Kn = dot(C, k_ref[...]), dot(N, k_ref[...])
    Qc, Qn = dot(C, q_ref[...]), dot(N, q_ref[...])
    Vc, Vn = dot(C, v_ref[...]), dot(N, v_ref[...])
    Kcs = Kc.astype(jnp.bfloat16).astype(jnp.float32)
    Kns = Kn.astype(jnp.bfloat16).astype(jnp.float32)
    Qcs = Qc.astype(jnp.bfloat16).astype(jnp.float32)
    Qns = Qn.astype(jnp.bfloat16).astype(jnp.float32)
    sq2 = jnp.float32(math.sqrt(2.0))
    s00 = jnp.sum(Kcs * Qcs, axis=1, keepdims=True) / sq2
    s01 = jnp.sum(Kcs * Qns, axis=1, keepdims=True) / sq2
    s10 = jnp.sum(Kns * Qcs, axis=1, keepdims=True) / sq2
    s11 = jnp.sum(Kns * Qns, axis=1, keepdims=True) / sq2
    m0 = jnp.maximum(s00, s10)
    e00 = jnp.exp(s00 - m0)
    e10 = jnp.exp(s10 - m0)
    d0 = e00 + e10
    m1 = jnp.maximum(s01, s11)
    e01 = jnp.exp(s01 - m1)
    e11 = jnp.exp(s11 - m1)
    d1 = e01 + e11
    f0 = (e00 / d0) * Vc + (e10 / d0) * Vn
    f1 = (e01 / d1) * Vc + (e11 / d1) * Vn
    w1 = w_ref[0:dim, :]
    w2 = w_ref[dim:2 * dim, :]
    dn2 = (((1,), (0,)), ((), ()))
    out = (lax.dot_general(f0, w1, dn2, preferred_element_type=jnp.float32,
                           precision=lax.Precision.DEFAULT)
           + lax.dot_general(f1, w2, dn2, preferred_element_type=jnp.float32,
                             precision=lax.Precision.DEFAULT)
           + b_ref[...])
    o_ref[:, 0, :] = out * (1.0 / (1.0 + jnp.exp(-1.702 * out)))


def _fuse_call(cls, new, K, Q, V, W_fuse, b2):
    bs, dim = cls.shape
    return pl.pallas_call(
        _fuse_body,
        out_shape=jax.ShapeDtypeStruct((bs, 1, dim), jnp.float32),
    )(cls, new, K, Q, V, W_fuse, b2)


def kernel(tokens, code_book, K, Q, V, W_fuse, b_fuse):
    bs, _, dim = tokens.shape
    book, cdim = code_book.shape
    cls = tokens[:, 0, :]
    t2 = cls.reshape(-1, cdim)
    t = _l2n(t2)
    code = _l2n(code_book)
    tsq = jnp.sum(t ** 2, axis=1, keepdims=True)
    csq = jnp.sum(code ** 2, axis=1)[None, :]
    idx = _argmin_call(-2.0 * t, code, tsq, csq)
    table = jnp.concatenate([code_book, code_book], axis=1)
    pair = _make_gather(book, 2 * cdim, t2.shape[0], 2 * cdim)(table, idx)
    new = pair[:, :cdim].reshape(bs, dim)
    return _fuse_call(cls, new, K, Q, V, W_fuse, b_fuse.reshape(1, dim))


# speedup vs baseline: 1.2561x; 1.0215x over previous
import functools
import math

import jax
import jax.numpy as jnp
from jax import lax
from jax.experimental import pallas as pl
from jax.experimental.pallas import tpu as pltpu
from jax.experimental.pallas import tpu_sc as plsc

_EPS = 1e-12
_RB = 3072
_CB = 2048
_LANES = 128

_NC = 2
_NS = 16
_NW = _NC * _NS


def _argmin_body(tm2_ref, c_ref, tsq_ref, csq_ref, val_ref, idx_ref,
                 rv_ref, ri_ref, *, nblk):
    jc = pl.program_id(1)

    def chunked(g):
        tsq = tsq_ref[...]
        nch = _CB // _LANES
        lv = li = None
        for ch in range(nch):
            s = lax.dot_general(
                tm2_ref[...], c_ref[ch * _LANES:(ch + 1) * _LANES, :],
                (((1,), (1,)), ((), ())),
                preferred_element_type=jnp.float32,
                precision=lax.Precision.DEFAULT)
            csq = csq_ref[:, pl.ds(g * _CB + ch * _LANES, _LANES)]
            d = (tsq + csq) + s
            j = lax.broadcasted_iota(jnp.int32, d.shape, 1) + (
                g * _CB + ch * _LANES)
            if lv is None:
                lv, li = d, j
            else:
                upd = d < lv
                lv = jnp.where(upd, d, lv)
                li = jnp.where(upd, j, li)
        upd = lv < rv_ref[...]
        rv_ref[...] = jnp.where(upd, lv, rv_ref[...])
        ri_ref[...] = jnp.where(upd, li, ri_ref[...])

    @pl.when(jc == 0)
    def _prologue():
        rv_ref[...] = jnp.full((_RB, _LANES), jnp.inf, jnp.float32)
        ri_ref[...] = jnp.zeros((_RB, _LANES), jnp.int32)

    chunked(jc)

    @pl.when(jc == nblk - 1)
    def _finalize():
        rv = rv_ref[...]
        ri = ri_ref[...]
        m = jnp.min(rv, axis=1, keepdims=True)
        cand = jnp.where(rv == m, ri, jnp.int32(2 ** 30))
        val_ref[...] = m
        idx_ref[...] = jnp.min(cand, axis=1, keepdims=True)


def _argmin_call(t, code, tsq, csq):
    rows, cdim = t.shape
    book = code.shape[0]
    nblk = book // _CB
    grid = (rows // _RB, nblk)
    _, idx = pl.pallas_call(
        functools.partial(_argmin_body, nblk=nblk),
        grid=grid,
        in_specs=[
            pl.BlockSpec((_RB, cdim), lambda i, j: (i, 0)),
            pl.BlockSpec((_CB, cdim), lambda i, j: (j, 0)),
            pl.BlockSpec((_RB, 1), lambda i, j: (i, 0)),
            pl.BlockSpec((1, book), lambda i, j: (0, 0)),
        ],
        out_specs=[
            pl.BlockSpec((_RB, 1), lambda i, j: (i, 0)),
            pl.BlockSpec((_RB, 1), lambda i, j: (i, 0)),
        ],
        out_shape=[
            jax.ShapeDtypeStruct((rows, 1), jnp.float32),
            jax.ShapeDtypeStruct((rows, 1), jnp.int32),
        ],
        scratch_shapes=[
            pltpu.VMEM((_RB, _LANES), jnp.float32),
            pltpu.VMEM((_RB, _LANES), jnp.int32),
        ],
    )(t, code, tsq, csq)
    return idx.reshape(rows)


def _l2n(x):
    n = jnp.sqrt(jnp.sum(x * x, axis=-1, keepdims=True))
    return x / jnp.maximum(n, _EPS)


@functools.lru_cache(maxsize=None)
def _make_gather(nrows_tbl, width, rows, out_width):
    bpw = rows // _NW
    mesh = plsc.VectorSubcoreMesh(core_axis_name="c", subcore_axis_name="s")

    @functools.partial(
        pl.kernel,
        mesh=mesh,
        out_type=jax.ShapeDtypeStruct((rows, out_width), jnp.float32),
        scratch_types=[
            pltpu.VMEM((bpw,), jnp.int32),
            pltpu.VMEM((bpw, width), jnp.float32),
            pltpu.SemaphoreType.DMA,
        ],
    )
    def gather_kernel(table_hbm, idx_hbm, out_hbm, idx_v, rows_v, sem):
        wid = lax.axis_index("s") * _NC + lax.axis_index("c")
        base = wid * bpw
        pltpu.sync_copy(idx_hbm.at[pl.ds(base, bpw)], idx_v)
        pltpu.async_copy(table_hbm.at[idx_v], rows_v, sem).wait()
        if out_width == width:
            pltpu.sync_copy(rows_v, out_hbm.at[pl.ds(base, bpw)])
        else:
            pltpu.sync_copy(rows_v.at[:, 0:out_width],
                            out_hbm.at[pl.ds(base, bpw)])

    return gather_kernel


def _fuse_body(cls_ref, new_ref, k_ref, q_ref, v_ref, w_ref, b_ref, o_ref):
    C = cls_ref[...]
    N = new_ref[...]
    dim = C.shape[1]
    dn = (((1,), (1,)), ((), ()))

    def dot(a, b):
        return lax.dot_general(a, b, dn,
                               preferred_element_type=jnp.float32,
                               precision=lax.Precision.DEFAULT)

    Kc, Kn = dot(C, k_ref[...]), dot(N, k_ref[...])
    Qc, Qn = dot(C, q_ref[...]), dot(N, q_ref[...])
    Vc, Vn = dot(C, v_ref[...]), dot(N, v_ref[...])
    Kcs = Kc.astype(jnp.bfloat16).astype(jnp.float32)
    Kns = Kn.astype(jnp.bfloat16).astype(jnp.float32)
    Qcs = Qc.astype(jnp.bfloat16).astype(jnp.float32)
    Qns = Qn.astype(jnp.bfloat16).astype(jnp.float32)
    sq2 = jnp.float32(math.sqrt(2.0))
    s00 = jnp.sum(Kcs * Qcs, axis=1, keepdims=True) / sq2
    s01 = jnp.sum(Kcs * Qns, axis=1, keepdims=True) / sq2
    s10 = jnp.sum(Kns * Qcs, axis=1, keepdims=True) / sq2
    s11 = jnp.sum(Kns * Qns, axis=1, keepdims=True) / sq2
    m0 = jnp.maximum(s00, s10)
    e00 = jnp.exp(s00 - m0)
    e10 = jnp.exp(s10 - m0)
    d0 = e00 + e10
    m1 = jnp.maximum(s01, s11)
    e01 = jnp.exp(s01 - m1)
    e11 = jnp.exp(s11 - m1)
    d1 = e01 + e11
    f0 = (e00 / d0) * Vc + (e10 / d0) * Vn
    f1 = (e01 / d1) * Vc + (e11 / d1) * Vn
    w1 = w_ref[0:dim, :]
    w2 = w_ref[dim:2 * dim, :]
    dn2 = (((1,), (0,)), ((), ()))
    out = (lax.dot_general(f0, w1, dn2, preferred_element_type=jnp.float32,
                           precision=lax.Precision.DEFAULT)
           + lax.dot_general(f1, w2, dn2, preferred_element_type=jnp.float32,
                             precision=lax.Precision.DEFAULT)
           + b_ref[...])
    o_ref[:, 0, :] = out * (1.0 / (1.0 + jnp.exp(-1.702 * out)))


def _fuse_call(cls, new, K, Q, V, W_fuse, b2):
    bs, dim = cls.shape
    return pl.pallas_call(
        _fuse_body,
        out_shape=jax.ShapeDtypeStruct((bs, 1, dim), jnp.float32),
    )(cls, new, K, Q, V, W_fuse, b2)


def kernel(tokens, code_book, K, Q, V, W_fuse, b_fuse):
    bs, _, dim = tokens.shape
    book, cdim = code_book.shape
    cls = tokens[:, 0, :]
    t2 = cls.reshape(-1, cdim)
    t = _l2n(t2)
    code = _l2n(code_book)
    tsq = jnp.sum(t ** 2, axis=1, keepdims=True)
    csq = jnp.sum(code ** 2, axis=1)[None, :]
    idx = _argmin_call(-2.0 * t, code, tsq, csq)
    table = jnp.concatenate([code_book, code_book], axis=1)
    pair = _make_gather(book, 2 * cdim, t2.shape[0], 2 * cdim)(table, idx)
    new = pair[:, :cdim].reshape(bs, dim)
    return _fuse_call(cls, new, K, Q, V, W_fuse, b_fuse.reshape(1, dim))
